# baseline (speedup 1.0000x reference)
import jax
import jax.numpy as jnp
from jax import lax
from jax.experimental import pallas as pl
from jax.experimental.pallas import tpu as pltpu
from jax.experimental.pallas import tpu_sc as plsc

BATCH = 16384
EMB = 32
N_USERS = 1000000
N_TOURS = 100000

_info = plsc.get_sparse_core_info()
_NC = _info.num_cores
_NS = _info.num_subcores
_NW = _NC * _NS
_BPW = BATCH // _NW

_RBLK = 4096


def _gather_body(users_hbm, tours_hbm, ut_hbm, tt_hbm, uout_hbm, tout_hbm,
                 uidx, tidx, urows, trows, sem_u, sem_t):
    wid = lax.axis_index("s") * _NC + lax.axis_index("c")
    base = wid * _BPW
    pltpu.sync_copy(users_hbm.at[pl.ds(base, _BPW)], uidx)
    pltpu.sync_copy(tours_hbm.at[pl.ds(base, _BPW)], tidx)
    cu = pltpu.async_copy(ut_hbm.at[uidx], urows, sem_u)
    ct = pltpu.async_copy(tt_hbm.at[tidx], trows, sem_t)
    cu.wait()
    ct.wait()
    pltpu.sync_copy(urows, uout_hbm.at[pl.ds(base, _BPW)])
    pltpu.sync_copy(trows, tout_hbm.at[pl.ds(base, _BPW)])


def _reduce_body(u_ref, t_ref, w_ref, b_ref, o_ref):
    wu = w_ref[0, :EMB]
    wt = w_ref[0, EMB:]
    o_ref[...] = (jnp.sum(u_ref[...] * wu, axis=1, keepdims=True)
                  + jnp.sum(t_ref[...] * wt, axis=1, keepdims=True)
                  + b_ref[0])


@jax.jit
def kernel(users, tours, user_table, tour_table, W, b):
    run = pl.kernel(
        _gather_body,
        out_type=(jax.ShapeDtypeStruct((BATCH, EMB), jnp.float32),
                  jax.ShapeDtypeStruct((BATCH, EMB), jnp.float32)),
        mesh=plsc.VectorSubcoreMesh(core_axis_name="c", subcore_axis_name="s"),
        compiler_params=pltpu.CompilerParams(
            needs_layout_passes=False, use_tc_tiling_on_sc=False),
        scratch_types=[
            pltpu.VMEM((_BPW,), jnp.int32),
            pltpu.VMEM((_BPW,), jnp.int32),
            pltpu.VMEM((_BPW, EMB), jnp.float32),
            pltpu.VMEM((_BPW, EMB), jnp.float32),
            pltpu.SemaphoreType.DMA,
            pltpu.SemaphoreType.DMA,
        ],
    )
    ug, tg = run(users.astype(jnp.int32), tours.astype(jnp.int32),
                 user_table, tour_table)

    out = pl.pallas_call(
        _reduce_body,
        grid=(BATCH // _RBLK,),
        in_specs=[
            pl.BlockSpec((_RBLK, EMB), lambda i: (i, 0)),
            pl.BlockSpec((_RBLK, EMB), lambda i: (i, 0)),
            pl.BlockSpec((1, 2 * EMB), lambda i: (0, 0)),
            pl.BlockSpec(memory_space=pltpu.SMEM),
        ],
        out_specs=pl.BlockSpec((_RBLK, 1), lambda i: (i, 0)),
        out_shape=jax.ShapeDtypeStruct((BATCH, 1), jnp.float32),
    )(ug, tg, W, b)
    return out


# speedup vs baseline: 7.8830x; 7.8830x over previous
import functools

import jax
import jax.numpy as jnp
from jax import lax
from jax.experimental import pallas as pl
from jax.experimental.pallas import tpu as pltpu
from jax.experimental.pallas import tpu_sc as plsc

BATCH = 16384
EMB = 32
N_USERS = 1000000
N_TOURS = 100000

_info = plsc.get_sparse_core_info()
_NC = _info.num_cores
_NS = _info.num_subcores
_L = _info.num_lanes
_NW = _NC * _NS
_BPW = BATCH // _NW

_UCHUNK = 131072
_TCHUNK = 51200


def _score_body(tT_ref, w_ref, b_ref, out_ref):
    out_ref[...] = jnp.sum(tT_ref[...] * w_ref[...], axis=0) + b_ref[0]


def _scores(tT, wcol, bias, n, chunk):
    grid = (n + chunk - 1) // chunk
    return pl.pallas_call(
        _score_body,
        grid=(grid,),
        in_specs=[
            pl.BlockSpec((EMB, chunk), lambda i: (0, i)),
            pl.BlockSpec((EMB, 1), lambda i: (0, 0)),
            pl.BlockSpec(memory_space=pltpu.SMEM),
        ],
        out_specs=pl.BlockSpec((chunk,), lambda i: (i,)),
        out_shape=jax.ShapeDtypeStruct((n,), jnp.float32),
    )(tT, wcol, bias)


def _gather_body(users_hbm, tours_hbm, us_hbm, ts_hbm, out_hbm,
                 uidx, tidx, uval, tval, outv, sem_u, sem_t):
    wid = lax.axis_index("s") * _NC + lax.axis_index("c")
    base = wid * _BPW
    pltpu.sync_copy(users_hbm.at[pl.ds(base, _BPW)], uidx)
    pltpu.sync_copy(tours_hbm.at[pl.ds(base, _BPW)], tidx)
    cu = pltpu.async_copy(us_hbm.at[uidx], uval, sem_u)
    ct = pltpu.async_copy(ts_hbm.at[tidx], tval, sem_t)
    cu.wait()
    ct.wait()

    def group(g, carry):
        sl = pl.ds(g * _L, _L)
        outv[sl] = uval[sl] + tval[sl]
        return carry

    lax.fori_loop(0, _BPW // _L, group, 0)
    pltpu.sync_copy(outv, out_hbm.at[pl.ds(base, _BPW)])


@jax.jit
def kernel(users, tours, user_table, tour_table, W, b):
    wu = W[0, :EMB].reshape(EMB, 1)
    wt = W[0, EMB:].reshape(EMB, 1)
    zero = jnp.zeros((1,), jnp.float32)
    uscore = _scores(user_table.T, wu, zero, N_USERS, _UCHUNK)
    tscore = _scores(tour_table.T, wt, b, N_TOURS, _TCHUNK)

    run = pl.kernel(
        _gather_body,
        out_type=jax.ShapeDtypeStruct((BATCH,), jnp.float32),
        mesh=plsc.VectorSubcoreMesh(core_axis_name="c", subcore_axis_name="s"),
        compiler_params=pltpu.CompilerParams(
            needs_layout_passes=False, use_tc_tiling_on_sc=False),
        scratch_types=[
            pltpu.VMEM((_BPW,), jnp.int32),
            pltpu.VMEM((_BPW,), jnp.int32),
            pltpu.VMEM((_BPW,), jnp.float32),
            pltpu.VMEM((_BPW,), jnp.float32),
            pltpu.VMEM((_BPW,), jnp.float32),
            pltpu.SemaphoreType.DMA,
            pltpu.SemaphoreType.DMA,
        ],
    )
    out = run(users.astype(jnp.int32), tours.astype(jnp.int32), uscore, tscore)
    return out.reshape(BATCH, 1)
